# Initial kernel scaffold; baseline (speedup 1.0000x reference)
import jax
import jax.numpy as jnp
from jax.experimental import pallas as pl


def kernel(x_user, x_item, edge_index_u2i, edge_index_i2u, W_l1_u2i, b_l1_u2i, W_r1_u2i, W_l1_i2u, b_l1_i2u, W_r1_i2u, W_l2_u2i, b_l2_u2i, W_r2_u2i, W_l2_i2u, b_l2_i2u, W_r2_i2u):
    raise NotImplementedError("write your pallas kernel here")


# speedup vs baseline: 5.5171x; 5.5171x over previous
import functools

import jax
import jax.numpy as jnp
from jax import lax
from jax.experimental import pallas as pl
from jax.experimental.pallas import tpu as pltpu
from jax.experimental.pallas import tpu_sc as plsc

N_USER = 10000
N_ITEM = 10000
E = 320000
D_IN = 128
H = 32

NC = 2
NS = 16
NW = NC * NS
CHUNK = 128
ROWS = 80
E_PAD = NW * CHUNK * ROWS
N_PAD = 10240
ZROWS = N_PAD // NS


def _mm_body(x_ref, w_ref, o_ref):
    o_ref[...] = jnp.dot(x_ref[...], w_ref[...],
                         preferred_element_type=jnp.float32)


def _matmul(x, w):
    return pl.pallas_call(
        _mm_body,
        out_shape=jax.ShapeDtypeStruct((x.shape[0], w.shape[1]), jnp.float32),
    )(x, w)


def _combine_body(parts_ref, cnt_ref, r_ref, b_ref, wcat_ref, o_ref):
    agg = parts_ref[0] + parts_ref[1]
    cnt = cnt_ref[0, :, :1] + cnt_ref[1, :, :1]
    h = jnp.maximum(agg / jnp.maximum(cnt, 1.0) + b_ref[...] + r_ref[...], 0.0)
    if wcat_ref is None:
        o_ref[...] = h
    else:
        o_ref[...] = jnp.dot(h, wcat_ref[...],
                             preferred_element_type=jnp.float32)


def _combine_project(parts, cnts, r, b, wcat):
    return pl.pallas_call(
        _combine_body,
        out_shape=jax.ShapeDtypeStruct((r.shape[0], wcat.shape[1]),
                                       jnp.float32),
    )(parts, cnts, r, b.reshape(1, H), wcat)


def _final_body(parts_ref, cnt_ref, r_ref, b_ref, o_ref):
    _combine_body(parts_ref, cnt_ref, r_ref, b_ref, None, o_ref)


def _combine_final(parts, cnts, r, b):
    return pl.pallas_call(
        _final_body,
        out_shape=jax.ShapeDtypeStruct((r.shape[0], H), jnp.float32),
    )(parts, cnts, r, b.reshape(1, H))


def _edge_pass(with_counts):
    mesh = plsc.VectorSubcoreMesh(core_axis_name="c", subcore_axis_name="s",
                                  num_cores=NC, num_subcores=NS)

    out_type = [
        jax.ShapeDtypeStruct((NC, N_PAD, H), jnp.float32),
        jax.ShapeDtypeStruct((NC, N_PAD, H), jnp.float32),
    ]
    scratch = [
        pltpu.VMEM_SHARED((N_PAD, H), jnp.float32),
        pltpu.VMEM_SHARED((N_PAD, H), jnp.float32),
        pltpu.VMEM((ROWS, CHUNK), jnp.int32),
        pltpu.VMEM((ROWS, CHUNK), jnp.int32),
        pltpu.VMEM((CHUNK, H), jnp.float32),
        pltpu.SemaphoreType.DMA,
    ]
    if with_counts:
        out_type += [
            jax.ShapeDtypeStruct((NC, N_PAD, 16), jnp.float32),
            jax.ShapeDtypeStruct((NC, N_PAD, 16), jnp.float32),
        ]
        scratch += [
            pltpu.VMEM_SHARED((N_PAD, 16), jnp.float32),
            pltpu.VMEM_SHARED((N_PAD, 16), jnp.float32),
            pltpu.VMEM((CHUNK, 16), jnp.float32),
        ]

    @functools.partial(
        pl.kernel, out_type=out_type, mesh=mesh, scratch_types=scratch,
        compiler_params=pltpu.CompilerParams(use_tc_tiling_on_sc=False))
    def k(*refs):
        if with_counts:
            (p_u2i, p_i2u, si_u2i, di_u2i, si_i2u, di_i2u, zeros_h, zeros_s,
             ones_h, agg_i_out, agg_u_out, cnt_i_out, cnt_u_out,
             acc_a, acc_b, sidx, didx, rows, sem,
             cnt_a, cnt_b, ones_v) = refs
        else:
            (p_u2i, p_i2u, si_u2i, di_u2i, si_i2u, di_i2u, zeros_h,
             agg_i_out, agg_u_out,
             acc_a, acc_b, sidx, didx, rows, sem) = refs

        cid = lax.axis_index("c")
        sid = lax.axis_index("s")
        wid = sid * NC + cid

        zsl = pl.ds(sid * ZROWS, ZROWS)
        pltpu.sync_copy(zeros_h, acc_a.at[zsl])
        pltpu.sync_copy(zeros_h, acc_b.at[zsl])
        if with_counts:
            pltpu.sync_copy(zeros_s, cnt_a.at[zsl])
            pltpu.sync_copy(zeros_s, cnt_b.at[zsl])
            pltpu.sync_copy(ones_h, ones_v)
        plsc.subcore_barrier()

        def do_rel(p_tab, si_hbm, di_hbm, acc_sh, cnt_sh):
            isl = pl.ds(wid * ROWS, ROWS)
            pltpu.sync_copy(si_hbm.at[isl], sidx)
            pltpu.sync_copy(di_hbm.at[isl], didx)

            def step(j, carry):
                pltpu.async_copy(p_tab.at[sidx.at[j]], rows, sem).wait()
                pltpu.sync_copy(rows, acc_sh.at[didx.at[j]], add=True)
                if cnt_sh is not None:
                    pltpu.sync_copy(ones_v, cnt_sh.at[didx.at[j]], add=True)
                return carry

            lax.fori_loop(0, ROWS, step, 0)

        do_rel(p_u2i, si_u2i, di_u2i, acc_a, cnt_a if with_counts else None)
        do_rel(p_i2u, si_i2u, di_i2u, acc_b, cnt_b if with_counts else None)
        plsc.subcore_barrier()

        pltpu.sync_copy(acc_a.at[zsl], agg_i_out.at[cid, zsl])
        pltpu.sync_copy(acc_b.at[zsl], agg_u_out.at[cid, zsl])
        if with_counts:
            pltpu.sync_copy(cnt_a.at[zsl], cnt_i_out.at[cid, zsl])
            pltpu.sync_copy(cnt_b.at[zsl], cnt_u_out.at[cid, zsl])

    return k


_edge_pass_l1 = _edge_pass(with_counts=True)
_edge_pass_l2 = _edge_pass(with_counts=False)


def _pad_rows(x):
    return jnp.pad(x, ((0, N_PAD - x.shape[0]), (0, 0)))


def _pad_idx(ix):
    return jnp.concatenate(
        [ix, jnp.full((E_PAD - E,), N_USER, jnp.int32)]).reshape(
            NW * ROWS, CHUNK)


def kernel(x_user, x_item, edge_index_u2i, edge_index_i2u,
           W_l1_u2i, b_l1_u2i, W_r1_u2i,
           W_l1_i2u, b_l1_i2u, W_r1_i2u,
           W_l2_u2i, b_l2_u2i, W_r2_u2i,
           W_l2_i2u, b_l2_i2u, W_r2_i2u):
    si_u2i = _pad_idx(edge_index_u2i[0])
    di_u2i = _pad_idx(edge_index_u2i[1])
    si_i2u = _pad_idx(edge_index_i2u[0])
    di_i2u = _pad_idx(edge_index_i2u[1])
    zeros_h = jnp.zeros((ZROWS, H), jnp.float32)
    zeros_s = jnp.zeros((ZROWS, 16), jnp.float32)
    ones_h = jnp.ones((CHUNK, 16), jnp.float32)

    cat_u = _matmul(x_user, jnp.concatenate([W_l1_u2i, W_r1_i2u], axis=1))
    cat_i = _matmul(x_item, jnp.concatenate([W_l1_i2u, W_r1_u2i], axis=1))
    p1_u = _pad_rows(cat_u[:, :H])
    p1_i = _pad_rows(cat_i[:, :H])

    agg1_i, agg1_u, cnt_i, cnt_u = _edge_pass_l1(
        p1_u, p1_i, si_u2i, di_u2i, si_i2u, di_i2u, zeros_h, zeros_s, ones_h)
    cnt_i = cnt_i[:, :N_ITEM]
    cnt_u = cnt_u[:, :N_USER]

    cat2_i = _combine_project(agg1_i[:, :N_ITEM], cnt_i, cat_i[:, H:],
                              b_l1_u2i,
                              jnp.concatenate([W_l2_i2u, W_r2_u2i], axis=1))
    cat2_u = _combine_project(agg1_u[:, :N_USER], cnt_u, cat_u[:, H:],
                              b_l1_i2u,
                              jnp.concatenate([W_l2_u2i, W_r2_i2u], axis=1))
    p2_u = _pad_rows(cat2_u[:, :H])
    p2_i = _pad_rows(cat2_i[:, :H])

    agg2_i, agg2_u = _edge_pass_l2(
        p2_u, p2_i, si_u2i, di_u2i, si_i2u, di_i2u, zeros_h)

    o_item = _combine_final(agg2_i[:, :N_ITEM], cnt_i, cat2_i[:, H:],
                            b_l2_u2i)
    o_user = _combine_final(agg2_u[:, :N_USER], cnt_u, cat2_u[:, H:],
                            b_l2_i2u)
    return (o_user, o_item)


# speedup vs baseline: 6.0117x; 1.0897x over previous
import functools

import jax
import jax.numpy as jnp
from jax import lax
from jax.experimental import pallas as pl
from jax.experimental.pallas import tpu as pltpu
from jax.experimental.pallas import tpu_sc as plsc

N_USER = 10000
N_ITEM = 10000
E = 320000
D_IN = 128
H = 32

NC = 2
NS = 16
NW = NC * NS
CHUNK = 128
ROWS = 80
E_PAD = NW * CHUNK * ROWS
N_PAD = 10240
ZROWS = N_PAD // NS


def _mm_body(x_ref, w_ref, o_ref):
    o_ref[...] = jnp.dot(x_ref[...], w_ref[...],
                         preferred_element_type=jnp.float32)


def _matmul(x, w):
    return pl.pallas_call(
        _mm_body,
        out_shape=jax.ShapeDtypeStruct((x.shape[0], w.shape[1]), jnp.float32),
    )(x, w)


def _combine_body(parts_ref, cnt_ref, r_ref, b_ref, wcat_ref, o_ref):
    agg = parts_ref[0] + parts_ref[1]
    cnt = cnt_ref[0, :, :1] + cnt_ref[1, :, :1]
    h = jnp.maximum(agg / jnp.maximum(cnt, 1.0) + b_ref[...] + r_ref[...], 0.0)
    if wcat_ref is None:
        o_ref[...] = h
    else:
        o_ref[...] = jnp.dot(h, wcat_ref[...],
                             preferred_element_type=jnp.float32)


def _combine_project(parts, cnts, r, b, wcat):
    return pl.pallas_call(
        _combine_body,
        out_shape=jax.ShapeDtypeStruct((r.shape[0], wcat.shape[1]),
                                       jnp.float32),
    )(parts, cnts, r, b.reshape(1, H), wcat)


def _final_body(parts_ref, cnt_ref, r_ref, b_ref, o_ref):
    _combine_body(parts_ref, cnt_ref, r_ref, b_ref, None, o_ref)


def _combine_final(parts, cnts, r, b):
    return pl.pallas_call(
        _final_body,
        out_shape=jax.ShapeDtypeStruct((r.shape[0], H), jnp.float32),
    )(parts, cnts, r, b.reshape(1, H))


def _edge_pass(with_counts):
    mesh = plsc.VectorSubcoreMesh(core_axis_name="c", subcore_axis_name="s",
                                  num_cores=NC, num_subcores=NS)

    out_type = [
        jax.ShapeDtypeStruct((NC, N_PAD, H), jnp.float32),
        jax.ShapeDtypeStruct((NC, N_PAD, H), jnp.float32),
    ]
    scratch = [
        pltpu.VMEM_SHARED((N_PAD, H), jnp.float32),
        pltpu.VMEM_SHARED((N_PAD, H), jnp.float32),
        pltpu.VMEM((ROWS, CHUNK), jnp.int32),
        pltpu.VMEM((ROWS, CHUNK), jnp.int32),
        pltpu.VMEM((2, CHUNK, H), jnp.float32),
        pltpu.SemaphoreType.DMA,
        pltpu.SemaphoreType.DMA,
        pltpu.SemaphoreType.DMA,
        pltpu.SemaphoreType.DMA,
    ]
    if with_counts:
        out_type += [
            jax.ShapeDtypeStruct((NC, N_PAD, 16), jnp.float32),
            jax.ShapeDtypeStruct((NC, N_PAD, 16), jnp.float32),
        ]
        scratch += [
            pltpu.VMEM_SHARED((N_PAD, 16), jnp.float32),
            pltpu.VMEM_SHARED((N_PAD, 16), jnp.float32),
            pltpu.VMEM((CHUNK, 16), jnp.float32),
            pltpu.SemaphoreType.DMA,
            pltpu.SemaphoreType.DMA,
        ]

    @functools.partial(
        pl.kernel, out_type=out_type, mesh=mesh, scratch_types=scratch,
        compiler_params=pltpu.CompilerParams(use_tc_tiling_on_sc=False))
    def k(*refs):
        if with_counts:
            (p_u2i, p_i2u, si_u2i, di_u2i, si_i2u, di_i2u, zeros_h, zeros_s,
             ones_h, agg_i_out, agg_u_out, cnt_i_out, cnt_u_out,
             acc_a, acc_b, sidx, didx, rows, gsem_a, gsem_b, ssem_a, ssem_b,
             cnt_a, cnt_b, ones_v, osem_a, osem_b) = refs
        else:
            (p_u2i, p_i2u, si_u2i, di_u2i, si_i2u, di_i2u, zeros_h,
             agg_i_out, agg_u_out,
             acc_a, acc_b, sidx, didx, rows, gsem_a, gsem_b, ssem_a,
             ssem_b) = refs

        cid = lax.axis_index("c")
        sid = lax.axis_index("s")
        wid = sid * NC + cid

        zsl = pl.ds(sid * ZROWS, ZROWS)
        pltpu.sync_copy(zeros_h, acc_a.at[zsl])
        pltpu.sync_copy(zeros_h, acc_b.at[zsl])
        if with_counts:
            pltpu.sync_copy(zeros_s, cnt_a.at[zsl])
            pltpu.sync_copy(zeros_s, cnt_b.at[zsl])
            pltpu.sync_copy(ones_h, ones_v)
        plsc.subcore_barrier()

        def do_rel(p_tab, si_hbm, di_hbm, acc_sh, cnt_sh):
            isl = pl.ds(wid * ROWS, ROWS)
            pltpu.sync_copy(si_hbm.at[isl], sidx)
            pltpu.sync_copy(di_hbm.at[isl], didx)

            def g_issue(j, b, sem):
                pltpu.async_copy(p_tab.at[sidx.at[j]], rows.at[b], sem)

            def g_wait(b, sem):
                pltpu.make_async_copy(p_tab.at[sidx.at[0]], rows.at[b],
                                      sem).wait()

            def s_issue(j, b, sem):
                pltpu.async_copy(rows.at[b], acc_sh.at[didx.at[j]], sem,
                                 add=True)

            def s_wait(b, sem):
                pltpu.make_async_copy(rows.at[b], acc_sh.at[didx.at[0]],
                                      sem).wait()

            def o_issue(j, sem):
                pltpu.async_copy(ones_v, cnt_sh.at[didx.at[j]], sem, add=True)

            def o_wait(sem):
                pltpu.make_async_copy(ones_v, cnt_sh.at[didx.at[0]],
                                      sem).wait()

            half = ROWS // 2
            g_issue(0, 0, gsem_a)

            def pair(i, carry):
                j0 = 2 * i
                g_wait(0, gsem_a)

                @pl.when(i > 0)
                def _():
                    s_wait(1, ssem_b)
                    if cnt_sh is not None:
                        o_wait(osem_a)

                g_issue(j0 + 1, 1, gsem_b)
                s_issue(j0, 0, ssem_a)
                if cnt_sh is not None:
                    o_issue(j0, osem_a)

                g_wait(1, gsem_b)
                s_wait(0, ssem_a)

                if cnt_sh is not None:
                    @pl.when(i > 0)
                    def _():
                        o_wait(osem_b)

                @pl.when(i < half - 1)
                def _():
                    g_issue(j0 + 2, 0, gsem_a)

                s_issue(j0 + 1, 1, ssem_b)
                if cnt_sh is not None:
                    o_issue(j0 + 1, osem_b)
                return carry

            lax.fori_loop(0, half, pair, 0)
            s_wait(1, ssem_b)
            if cnt_sh is not None:
                o_wait(osem_a)
                o_wait(osem_b)

        do_rel(p_u2i, si_u2i, di_u2i, acc_a, cnt_a if with_counts else None)
        do_rel(p_i2u, si_i2u, di_i2u, acc_b, cnt_b if with_counts else None)
        plsc.subcore_barrier()

        pltpu.sync_copy(acc_a.at[zsl], agg_i_out.at[cid, zsl])
        pltpu.sync_copy(acc_b.at[zsl], agg_u_out.at[cid, zsl])
        if with_counts:
            pltpu.sync_copy(cnt_a.at[zsl], cnt_i_out.at[cid, zsl])
            pltpu.sync_copy(cnt_b.at[zsl], cnt_u_out.at[cid, zsl])

    return k


_edge_pass_l1 = _edge_pass(with_counts=True)
_edge_pass_l2 = _edge_pass(with_counts=False)


def _pad_rows(x):
    return jnp.pad(x, ((0, N_PAD - x.shape[0]), (0, 0)))


def _pad_idx(ix):
    return jnp.concatenate(
        [ix, jnp.full((E_PAD - E,), N_USER, jnp.int32)]).reshape(
            NW * ROWS, CHUNK)


def kernel(x_user, x_item, edge_index_u2i, edge_index_i2u,
           W_l1_u2i, b_l1_u2i, W_r1_u2i,
           W_l1_i2u, b_l1_i2u, W_r1_i2u,
           W_l2_u2i, b_l2_u2i, W_r2_u2i,
           W_l2_i2u, b_l2_i2u, W_r2_i2u):
    si_u2i = _pad_idx(edge_index_u2i[0])
    di_u2i = _pad_idx(edge_index_u2i[1])
    si_i2u = _pad_idx(edge_index_i2u[0])
    di_i2u = _pad_idx(edge_index_i2u[1])
    zeros_h = jnp.zeros((ZROWS, H), jnp.float32)
    zeros_s = jnp.zeros((ZROWS, 16), jnp.float32)
    ones_h = jnp.ones((CHUNK, 16), jnp.float32)

    cat_u = _matmul(x_user, jnp.concatenate([W_l1_u2i, W_r1_i2u], axis=1))
    cat_i = _matmul(x_item, jnp.concatenate([W_l1_i2u, W_r1_u2i], axis=1))
    p1_u = _pad_rows(cat_u[:, :H])
    p1_i = _pad_rows(cat_i[:, :H])

    agg1_i, agg1_u, cnt_i, cnt_u = _edge_pass_l1(
        p1_u, p1_i, si_u2i, di_u2i, si_i2u, di_i2u, zeros_h, zeros_s, ones_h)
    cnt_i = cnt_i[:, :N_ITEM]
    cnt_u = cnt_u[:, :N_USER]

    cat2_i = _combine_project(agg1_i[:, :N_ITEM], cnt_i, cat_i[:, H:],
                              b_l1_u2i,
                              jnp.concatenate([W_l2_i2u, W_r2_u2i], axis=1))
    cat2_u = _combine_project(agg1_u[:, :N_USER], cnt_u, cat_u[:, H:],
                              b_l1_i2u,
                              jnp.concatenate([W_l2_u2i, W_r2_i2u], axis=1))
    p2_u = _pad_rows(cat2_u[:, :H])
    p2_i = _pad_rows(cat2_i[:, :H])

    agg2_i, agg2_u = _edge_pass_l2(
        p2_u, p2_i, si_u2i, di_u2i, si_i2u, di_i2u, zeros_h)

    o_item = _combine_final(agg2_i[:, :N_ITEM], cnt_i, cat2_i[:, H:],
                            b_l2_u2i)
    o_user = _combine_final(agg2_u[:, :N_USER], cnt_u, cat2_u[:, H:],
                            b_l2_i2u)
    return (o_user, o_item)


# speedup vs baseline: 12.3806x; 2.0594x over previous
import functools

import jax
import jax.numpy as jnp
from jax import lax
from jax.experimental import pallas as pl
from jax.experimental.pallas import tpu as pltpu
from jax.experimental.pallas import tpu_sc as plsc

N_USER = 10000
N_ITEM = 10000
E = 320000
D_IN = 128
H = 32

NC = 2
NS = 16
NW = NC * NS
CHUNK = 128
ROWS = 80
E_PAD = NW * CHUNK * ROWS
N_PAD = 10240
ZROWS = N_PAD // NS


def _mm_body(x_ref, w_ref, o_ref):
    o_ref[...] = jnp.dot(x_ref[...], w_ref[...],
                         preferred_element_type=jnp.float32)


def _matmul(x, w):
    return pl.pallas_call(
        _mm_body,
        out_shape=jax.ShapeDtypeStruct((x.shape[0], w.shape[1]), jnp.float32),
    )(x, w)


def _combine_body(parts_ref, cnt_ref, r_ref, b_ref, wcat_ref, o_ref):
    agg = parts_ref[0] + parts_ref[1]
    cnt = cnt_ref[0, :, :1] + cnt_ref[1, :, :1]
    h = jnp.maximum(agg / jnp.maximum(cnt, 1.0) + b_ref[...] + r_ref[...], 0.0)
    if wcat_ref is None:
        o_ref[...] = h
    else:
        o_ref[...] = jnp.dot(h, wcat_ref[...],
                             preferred_element_type=jnp.float32)


def _combine_project(parts, cnts, r, b, wcat):
    return pl.pallas_call(
        _combine_body,
        out_shape=jax.ShapeDtypeStruct((r.shape[0], wcat.shape[1]),
                                       jnp.float32),
    )(parts, cnts, r, b.reshape(1, H), wcat)


def _final_body(parts_ref, cnt_ref, r_ref, b_ref, o_ref):
    _combine_body(parts_ref, cnt_ref, r_ref, b_ref, None, o_ref)


def _combine_final(parts, cnts, r, b):
    return pl.pallas_call(
        _final_body,
        out_shape=jax.ShapeDtypeStruct((r.shape[0], H), jnp.float32),
    )(parts, cnts, r, b.reshape(1, H))


def _edge_pass(with_counts):
    mesh = plsc.VectorSubcoreMesh(core_axis_name="c", subcore_axis_name="s",
                                  num_cores=NC, num_subcores=NS)

    out_type = [
        jax.ShapeDtypeStruct((NC, N_PAD, H), jnp.float32),
        jax.ShapeDtypeStruct((NC, N_PAD, H), jnp.float32),
    ]
    scratch = [
        pltpu.VMEM_SHARED((N_PAD, H), jnp.float32),
        pltpu.VMEM_SHARED((N_PAD, H), jnp.float32),
        pltpu.VMEM_SHARED((N_PAD, H), jnp.float32),
        pltpu.VMEM((ROWS, CHUNK), jnp.int32),
        pltpu.VMEM((ROWS, CHUNK), jnp.int32),
        pltpu.VMEM((2, CHUNK, H), jnp.float32),
        pltpu.SemaphoreType.DMA,
        pltpu.SemaphoreType.DMA,
        pltpu.SemaphoreType.DMA,
        pltpu.SemaphoreType.DMA,
    ]
    if with_counts:
        out_type += [
            jax.ShapeDtypeStruct((NC, N_PAD, 16), jnp.float32),
            jax.ShapeDtypeStruct((NC, N_PAD, 16), jnp.float32),
        ]
        scratch += [
            pltpu.VMEM_SHARED((N_PAD, 16), jnp.float32),
            pltpu.VMEM_SHARED((N_PAD, 16), jnp.float32),
            pltpu.VMEM((CHUNK, 16), jnp.float32),
            pltpu.SemaphoreType.DMA,
            pltpu.SemaphoreType.DMA,
        ]

    @functools.partial(
        pl.kernel, out_type=out_type, mesh=mesh, scratch_types=scratch,
        compiler_params=pltpu.CompilerParams(use_tc_tiling_on_sc=False))
    def k(*refs):
        if with_counts:
            (p_u2i, p_i2u, si_u2i, di_u2i, si_i2u, di_i2u, zeros_h, zeros_s,
             ones_h, agg_i_out, agg_u_out, cnt_i_out, cnt_u_out,
             acc_a, acc_b, tab, sidx, didx, rows,
             gsem_a, gsem_b, ssem_a, ssem_b,
             cnt_a, cnt_b, ones_v, osem_a, osem_b) = refs
        else:
            (p_u2i, p_i2u, si_u2i, di_u2i, si_i2u, di_i2u, zeros_h,
             agg_i_out, agg_u_out,
             acc_a, acc_b, tab, sidx, didx, rows,
             gsem_a, gsem_b, ssem_a, ssem_b) = refs

        cid = lax.axis_index("c")
        sid = lax.axis_index("s")
        wid = sid * NC + cid

        zsl = pl.ds(sid * ZROWS, ZROWS)
        pltpu.sync_copy(zeros_h, acc_a.at[zsl])
        pltpu.sync_copy(zeros_h, acc_b.at[zsl])
        pltpu.sync_copy(p_u2i.at[zsl], tab.at[zsl])
        if with_counts:
            pltpu.sync_copy(zeros_s, cnt_a.at[zsl])
            pltpu.sync_copy(zeros_s, cnt_b.at[zsl])
            pltpu.sync_copy(ones_h, ones_v)
        plsc.subcore_barrier()

        def do_rel(p_tab, si_hbm, di_hbm, acc_sh, cnt_sh):
            isl = pl.ds(wid * ROWS, ROWS)
            pltpu.sync_copy(si_hbm.at[isl], sidx)
            pltpu.sync_copy(di_hbm.at[isl], didx)

            def g_issue(j, b, sem):
                pltpu.async_copy(p_tab.at[sidx.at[j]], rows.at[b], sem)

            def g_wait(b, sem):
                pltpu.make_async_copy(p_tab.at[sidx.at[0]], rows.at[b],
                                      sem).wait()

            def s_issue(j, b, sem):
                pltpu.async_copy(rows.at[b], acc_sh.at[didx.at[j]], sem,
                                 add=True)

            def s_wait(b, sem):
                pltpu.make_async_copy(rows.at[b], acc_sh.at[didx.at[0]],
                                      sem).wait()

            def o_issue(j, sem):
                pltpu.async_copy(ones_v, cnt_sh.at[didx.at[j]], sem, add=True)

            def o_wait(sem):
                pltpu.make_async_copy(ones_v, cnt_sh.at[didx.at[0]],
                                      sem).wait()

            half = ROWS // 2
            g_issue(0, 0, gsem_a)

            def pair(i, carry):
                j0 = 2 * i
                g_wait(0, gsem_a)

                @pl.when(i > 0)
                def _():
                    s_wait(1, ssem_b)
                    if cnt_sh is not None:
                        o_wait(osem_a)

                g_issue(j0 + 1, 1, gsem_b)
                s_issue(j0, 0, ssem_a)
                if cnt_sh is not None:
                    o_issue(j0, osem_a)

                g_wait(1, gsem_b)
                s_wait(0, ssem_a)

                if cnt_sh is not None:
                    @pl.when(i > 0)
                    def _():
                        o_wait(osem_b)

                @pl.when(i < half - 1)
                def _():
                    g_issue(j0 + 2, 0, gsem_a)

                s_issue(j0 + 1, 1, ssem_b)
                if cnt_sh is not None:
                    o_issue(j0 + 1, osem_b)
                return carry

            lax.fori_loop(0, half, pair, 0)
            s_wait(1, ssem_b)
            if cnt_sh is not None:
                o_wait(osem_a)
                o_wait(osem_b)

        do_rel(tab, si_u2i, di_u2i, acc_a, cnt_a if with_counts else None)
        plsc.subcore_barrier()
        pltpu.sync_copy(p_i2u.at[zsl], tab.at[zsl])
        plsc.subcore_barrier()
        do_rel(tab, si_i2u, di_i2u, acc_b, cnt_b if with_counts else None)
        plsc.subcore_barrier()

        pltpu.sync_copy(acc_a.at[zsl], agg_i_out.at[cid, zsl])
        pltpu.sync_copy(acc_b.at[zsl], agg_u_out.at[cid, zsl])
        if with_counts:
            pltpu.sync_copy(cnt_a.at[zsl], cnt_i_out.at[cid, zsl])
            pltpu.sync_copy(cnt_b.at[zsl], cnt_u_out.at[cid, zsl])

    return k


_edge_pass_l1 = _edge_pass(with_counts=True)
_edge_pass_l2 = _edge_pass(with_counts=False)


def _pad_rows(x):
    return jnp.pad(x, ((0, N_PAD - x.shape[0]), (0, 0)))


def _pad_idx(ix):
    pad = N_USER + jnp.arange(E_PAD - E, dtype=jnp.int32) % (N_PAD - N_USER)
    return jnp.concatenate([ix, pad]).reshape(NW * ROWS, CHUNK)


def kernel(x_user, x_item, edge_index_u2i, edge_index_i2u,
           W_l1_u2i, b_l1_u2i, W_r1_u2i,
           W_l1_i2u, b_l1_i2u, W_r1_i2u,
           W_l2_u2i, b_l2_u2i, W_r2_u2i,
           W_l2_i2u, b_l2_i2u, W_r2_i2u):
    si_u2i = _pad_idx(edge_index_u2i[0])
    di_u2i = _pad_idx(edge_index_u2i[1])
    si_i2u = _pad_idx(edge_index_i2u[0])
    di_i2u = _pad_idx(edge_index_i2u[1])
    zeros_h = jnp.zeros((ZROWS, H), jnp.float32)
    zeros_s = jnp.zeros((ZROWS, 16), jnp.float32)
    ones_h = jnp.ones((CHUNK, 16), jnp.float32)

    cat_u = _matmul(x_user, jnp.concatenate([W_l1_u2i, W_r1_i2u], axis=1))
    cat_i = _matmul(x_item, jnp.concatenate([W_l1_i2u, W_r1_u2i], axis=1))
    p1_u = _pad_rows(cat_u[:, :H])
    p1_i = _pad_rows(cat_i[:, :H])

    agg1_i, agg1_u, cnt_i, cnt_u = _edge_pass_l1(
        p1_u, p1_i, si_u2i, di_u2i, si_i2u, di_i2u, zeros_h, zeros_s, ones_h)
    cnt_i = cnt_i[:, :N_ITEM]
    cnt_u = cnt_u[:, :N_USER]

    cat2_i = _combine_project(agg1_i[:, :N_ITEM], cnt_i, cat_i[:, H:],
                              b_l1_u2i,
                              jnp.concatenate([W_l2_i2u, W_r2_u2i], axis=1))
    cat2_u = _combine_project(agg1_u[:, :N_USER], cnt_u, cat_u[:, H:],
                              b_l1_i2u,
                              jnp.concatenate([W_l2_u2i, W_r2_i2u], axis=1))
    p2_u = _pad_rows(cat2_u[:, :H])
    p2_i = _pad_rows(cat2_i[:, :H])

    agg2_i, agg2_u = _edge_pass_l2(
        p2_u, p2_i, si_u2i, di_u2i, si_i2u, di_i2u, zeros_h)

    o_item = _combine_final(agg2_i[:, :N_ITEM], cnt_i, cat2_i[:, H:],
                            b_l2_u2i)
    o_user = _combine_final(agg2_u[:, :N_USER], cnt_u, cat2_u[:, H:],
                            b_l2_i2u)
    return (o_user, o_item)


# speedup vs baseline: 12.5631x; 1.0147x over previous
import functools

import jax
import jax.numpy as jnp
from jax import lax
from jax.experimental import pallas as pl
from jax.experimental.pallas import tpu as pltpu
from jax.experimental.pallas import tpu_sc as plsc

N_USER = 10000
N_ITEM = 10000
E = 320000
D_IN = 128
H = 32

NC = 2
NS = 16
NW = NC * NS
CHUNK = 128
ROWS = 80
DEPTH = 4
AHEAD = 2
E_PAD = NW * CHUNK * ROWS
N_PAD = 10240
ZROWS = N_PAD // NS


def _mm2_body(xa_ref, wa_ref, xb_ref, wb_ref, oa_ref, ob_ref):
    oa_ref[...] = jnp.dot(xa_ref[...], wa_ref[...],
                          preferred_element_type=jnp.float32)
    ob_ref[...] = jnp.dot(xb_ref[...], wb_ref[...],
                          preferred_element_type=jnp.float32)


def _matmul2(xa, wa, xb, wb):
    return pl.pallas_call(
        _mm2_body,
        out_shape=[
            jax.ShapeDtypeStruct((xa.shape[0], wa.shape[1]), jnp.float32),
            jax.ShapeDtypeStruct((xb.shape[0], wb.shape[1]), jnp.float32),
        ],
    )(xa, wa, xb, wb)


def _combine(parts_ref, cnt_ref, r_ref, b_ref, wcat_ref):
    agg = parts_ref[0] + parts_ref[1]
    cnt = cnt_ref[0, :, :1] + cnt_ref[1, :, :1]
    h = jnp.maximum(agg / jnp.maximum(cnt, 1.0) + b_ref[...] + r_ref[...], 0.0)
    if wcat_ref is None:
        return h
    return jnp.dot(h, wcat_ref[...], preferred_element_type=jnp.float32)


def _combine2_body(pa, ca, ra, ba, wa, pb, cb, rb, bb, wb, oa_ref, ob_ref):
    oa_ref[...] = _combine(pa, ca, ra, ba, wa)
    ob_ref[...] = _combine(pb, cb, rb, bb, wb)


def _final2_body(pa, ca, ra, ba, pb, cb, rb, bb, oa_ref, ob_ref):
    oa_ref[...] = _combine(pa, ca, ra, ba, None)
    ob_ref[...] = _combine(pb, cb, rb, bb, None)


def _combine_project2(pa, ca, ra, ba, wa, pb, cb, rb, bb, wb):
    return pl.pallas_call(
        _combine2_body,
        out_shape=[
            jax.ShapeDtypeStruct((ra.shape[0], wa.shape[1]), jnp.float32),
            jax.ShapeDtypeStruct((rb.shape[0], wb.shape[1]), jnp.float32),
        ],
        compiler_params=pltpu.CompilerParams(
            vmem_limit_bytes=100 * 1024 * 1024),
    )(pa, ca, ra, ba.reshape(1, H), wa, pb, cb, rb, bb.reshape(1, H), wb)


def _combine_final2(pa, ca, ra, ba, pb, cb, rb, bb):
    return pl.pallas_call(
        _final2_body,
        out_shape=[
            jax.ShapeDtypeStruct((ra.shape[0], H), jnp.float32),
            jax.ShapeDtypeStruct((rb.shape[0], H), jnp.float32),
        ],
        compiler_params=pltpu.CompilerParams(
            vmem_limit_bytes=100 * 1024 * 1024),
    )(pa, ca, ra, ba.reshape(1, H), pb, cb, rb, bb.reshape(1, H))


def _edge_pass(with_counts):
    mesh = plsc.VectorSubcoreMesh(core_axis_name="c", subcore_axis_name="s",
                                  num_cores=NC, num_subcores=NS)

    out_type = [
        jax.ShapeDtypeStruct((NC, N_PAD, H), jnp.float32),
        jax.ShapeDtypeStruct((NC, N_PAD, H), jnp.float32),
    ]
    scratch = [
        pltpu.VMEM_SHARED((N_PAD, H), jnp.float32),
        pltpu.VMEM_SHARED((N_PAD, H), jnp.float32),
        pltpu.VMEM_SHARED((N_PAD, H), jnp.float32),
        pltpu.VMEM((ROWS, CHUNK), jnp.int32),
        pltpu.VMEM((ROWS, CHUNK), jnp.int32),
        pltpu.VMEM((DEPTH, CHUNK, H), jnp.float32),
        [pltpu.SemaphoreType.DMA] * DEPTH,
        [pltpu.SemaphoreType.DMA] * DEPTH,
    ]
    if with_counts:
        out_type += [
            jax.ShapeDtypeStruct((NC, N_PAD, 16), jnp.float32),
            jax.ShapeDtypeStruct((NC, N_PAD, 16), jnp.float32),
        ]
        scratch += [
            pltpu.VMEM_SHARED((N_PAD, 16), jnp.float32),
            pltpu.VMEM_SHARED((N_PAD, 16), jnp.float32),
            pltpu.VMEM((CHUNK, 16), jnp.float32),
            [pltpu.SemaphoreType.DMA] * DEPTH,
        ]

    @functools.partial(
        pl.kernel, out_type=out_type, mesh=mesh, scratch_types=scratch,
        compiler_params=pltpu.CompilerParams(use_tc_tiling_on_sc=False))
    def k(*refs):
        if with_counts:
            (p_u2i, p_i2u, si_u2i, di_u2i, si_i2u, di_i2u, zeros_h, zeros_s,
             ones_h, agg_i_out, agg_u_out, cnt_i_out, cnt_u_out,
             acc_a, acc_b, tab, sidx, didx, rows, gsems, ssems,
             cnt_a, cnt_b, ones_v, osems) = refs
        else:
            (p_u2i, p_i2u, si_u2i, di_u2i, si_i2u, di_i2u, zeros_h,
             agg_i_out, agg_u_out,
             acc_a, acc_b, tab, sidx, didx, rows, gsems, ssems) = refs

        cid = lax.axis_index("c")
        sid = lax.axis_index("s")
        wid = sid * NC + cid

        gsem_a, gsem_b = gsems[0], gsems[1]
        ssem_a, ssem_b = ssems[0], ssems[1]
        if with_counts:
            osem_a, osem_b = osems[0], osems[1]

        zsl = pl.ds(sid * ZROWS, ZROWS)
        pltpu.sync_copy(zeros_h, acc_a.at[zsl])
        pltpu.sync_copy(zeros_h, acc_b.at[zsl])
        pltpu.sync_copy(p_u2i.at[zsl], tab.at[zsl])
        if with_counts:
            pltpu.sync_copy(zeros_s, cnt_a.at[zsl])
            pltpu.sync_copy(zeros_s, cnt_b.at[zsl])
            pltpu.sync_copy(ones_h, ones_v)
        plsc.subcore_barrier()

        def do_rel(p_tab, si_hbm, di_hbm, acc_sh, cnt_sh):
            isl = pl.ds(wid * ROWS, ROWS)
            pltpu.sync_copy(si_hbm.at[isl], sidx)
            pltpu.sync_copy(di_hbm.at[isl], didx)

            def g_issue(j, b, sem):
                pltpu.async_copy(p_tab.at[sidx.at[j]], rows.at[b], sem)

            def g_wait(b, sem):
                pltpu.make_async_copy(p_tab.at[sidx.at[0]], rows.at[b],
                                      sem).wait()

            def s_issue(j, b, sem):
                pltpu.async_copy(rows.at[b], acc_sh.at[didx.at[j]], sem,
                                 add=True)

            def s_wait(b, sem):
                pltpu.make_async_copy(rows.at[b], acc_sh.at[didx.at[0]],
                                      sem).wait()

            def o_issue(j, sem):
                pltpu.async_copy(ones_v, cnt_sh.at[didx.at[j]], sem, add=True)

            def o_wait(sem):
                pltpu.make_async_copy(ones_v, cnt_sh.at[didx.at[0]],
                                      sem).wait()

            half = ROWS // 2
            g_issue(0, 0, gsem_a)

            def pair(i, carry):
                j0 = 2 * i
                g_wait(0, gsem_a)

                @pl.when(i > 0)
                def _():
                    s_wait(1, ssem_b)
                    if cnt_sh is not None:
                        o_wait(osem_a)

                g_issue(j0 + 1, 1, gsem_b)
                s_issue(j0, 0, ssem_a)
                if cnt_sh is not None:
                    o_issue(j0, osem_a)

                g_wait(1, gsem_b)
                s_wait(0, ssem_a)

                if cnt_sh is not None:
                    @pl.when(i > 0)
                    def _():
                        o_wait(osem_b)

                @pl.when(i < half - 1)
                def _():
                    g_issue(j0 + 2, 0, gsem_a)

                s_issue(j0 + 1, 1, ssem_b)
                if cnt_sh is not None:
                    o_issue(j0 + 1, osem_b)
                return carry

            lax.fori_loop(0, half, pair, 0)
            s_wait(1, ssem_b)
            if cnt_sh is not None:
                o_wait(osem_a)
                o_wait(osem_b)

        do_rel(tab, si_u2i, di_u2i, acc_a, cnt_a if with_counts else None)
        plsc.subcore_barrier()
        pltpu.sync_copy(p_i2u.at[zsl], tab.at[zsl])
        plsc.subcore_barrier()
        do_rel(tab, si_i2u, di_i2u, acc_b, cnt_b if with_counts else None)
        plsc.subcore_barrier()

        pltpu.sync_copy(acc_a.at[zsl], agg_i_out.at[cid, zsl])
        pltpu.sync_copy(acc_b.at[zsl], agg_u_out.at[cid, zsl])
        if with_counts:
            pltpu.sync_copy(cnt_a.at[zsl], cnt_i_out.at[cid, zsl])
            pltpu.sync_copy(cnt_b.at[zsl], cnt_u_out.at[cid, zsl])

    return k


_edge_pass_l1 = _edge_pass(with_counts=True)
_edge_pass_l2 = _edge_pass(with_counts=False)


def _pad_rows(x):
    return jnp.pad(x, ((0, N_PAD - x.shape[0]), (0, 0)))


def _pad_idx(ix):
    pad = N_USER + jnp.arange(E_PAD - E, dtype=jnp.int32) % (N_PAD - N_USER)
    return jnp.concatenate([ix, pad]).reshape(NW * ROWS, CHUNK)


def kernel(x_user, x_item, edge_index_u2i, edge_index_i2u,
           W_l1_u2i, b_l1_u2i, W_r1_u2i,
           W_l1_i2u, b_l1_i2u, W_r1_i2u,
           W_l2_u2i, b_l2_u2i, W_r2_u2i,
           W_l2_i2u, b_l2_i2u, W_r2_i2u):
    si_u2i = _pad_idx(edge_index_u2i[0])
    di_u2i = _pad_idx(edge_index_u2i[1])
    si_i2u = _pad_idx(edge_index_i2u[0])
    di_i2u = _pad_idx(edge_index_i2u[1])
    zeros_h = jnp.zeros((ZROWS, H), jnp.float32)
    zeros_s = jnp.zeros((ZROWS, 16), jnp.float32)
    ones_h = jnp.ones((CHUNK, 16), jnp.float32)

    cat_u, cat_i = _matmul2(
        x_user, jnp.concatenate([W_l1_u2i, W_r1_i2u], axis=1),
        x_item, jnp.concatenate([W_l1_i2u, W_r1_u2i], axis=1))
    p1_u = _pad_rows(cat_u[:, :H])
    p1_i = _pad_rows(cat_i[:, :H])

    agg1_i, agg1_u, cnt_i, cnt_u = _edge_pass_l1(
        p1_u, p1_i, si_u2i, di_u2i, si_i2u, di_i2u, zeros_h, zeros_s, ones_h)
    cnt_i = cnt_i[:, :N_ITEM]
    cnt_u = cnt_u[:, :N_USER]

    cat2_i, cat2_u = _combine_project2(
        agg1_i[:, :N_ITEM], cnt_i, cat_i[:, H:], b_l1_u2i,
        jnp.concatenate([W_l2_i2u, W_r2_u2i], axis=1),
        agg1_u[:, :N_USER], cnt_u, cat_u[:, H:], b_l1_i2u,
        jnp.concatenate([W_l2_u2i, W_r2_i2u], axis=1))
    p2_u = _pad_rows(cat2_u[:, :H])
    p2_i = _pad_rows(cat2_i[:, :H])

    agg2_i, agg2_u = _edge_pass_l2(
        p2_u, p2_i, si_u2i, di_u2i, si_i2u, di_i2u, zeros_h)

    o_item, o_user = _combine_final2(
        agg2_i[:, :N_ITEM], cnt_i, cat2_i[:, H:], b_l2_u2i,
        agg2_u[:, :N_USER], cnt_u, cat2_u[:, H:], b_l2_i2u)
    return (o_user, o_item)


# speedup vs baseline: 13.8146x; 1.0996x over previous
import functools

import jax
import jax.numpy as jnp
from jax import lax
from jax.experimental import pallas as pl
from jax.experimental.pallas import tpu as pltpu
from jax.experimental.pallas import tpu_sc as plsc

N_USER = 10000
N_ITEM = 10000
E = 320000
D_IN = 128
H = 32

NC = 2
NS = 16
NW = NC * NS
CHUNK = 128
ROWS = 80
DEPTH = 4
AHEAD = 2
E_PAD = NW * CHUNK * ROWS
N_PAD = 10240
ZROWS = N_PAD // NS


_VMEM100 = pltpu.CompilerParams(vmem_limit_bytes=100 * 1024 * 1024)


def _proj4_body(xu, xi, wa, wb, wc, wd, o1, o2, o3, o4):
    o1[...] = jnp.dot(xu[...], wa[...], preferred_element_type=jnp.float32)
    o2[...] = jnp.dot(xu[...], wb[...], preferred_element_type=jnp.float32)
    o3[...] = jnp.dot(xi[...], wc[...], preferred_element_type=jnp.float32)
    o4[...] = jnp.dot(xi[...], wd[...], preferred_element_type=jnp.float32)


def _proj4(xu, wa, wb, xi, wc, wd):
    return pl.pallas_call(
        _proj4_body,
        out_shape=[jax.ShapeDtypeStruct((N_PAD, H), jnp.float32)] * 4,
        compiler_params=_VMEM100,
    )(xu, xi, wa, wb, wc, wd)


def _combine(parts_ref, cnt_ref, r_ref, b_ref):
    agg = parts_ref[0] + parts_ref[1]
    cnt = cnt_ref[0, :, :1] + cnt_ref[1, :, :1]
    return jnp.maximum(agg / jnp.maximum(cnt, 1.0) + b_ref[...] + r_ref[...],
                       0.0)


def _mid_body(p, c, r, b, wl, wr, o1, o2):
    h = _combine(p, c, r, b)
    o1[...] = jnp.dot(h, wl[...], preferred_element_type=jnp.float32)
    o2[...] = jnp.dot(h, wr[...], preferred_element_type=jnp.float32)


def _fin_body(p, c, r, b, o_ref):
    o_ref[...] = _combine(p, c, r, b)[:o_ref.shape[0]]


def _combine_project(p, c, r, b, wl, wr):
    return pl.pallas_call(
        _mid_body,
        out_shape=[jax.ShapeDtypeStruct((N_PAD, H), jnp.float32)] * 2,
        compiler_params=_VMEM100,
    )(p, c, r, b.reshape(1, H), wl, wr)


def _combine_final(p, c, r, b, n):
    return pl.pallas_call(
        _fin_body,
        out_shape=jax.ShapeDtypeStruct((n, H), jnp.float32),
        compiler_params=_VMEM100,
    )(p, c, r, b.reshape(1, H))


def _edge_pass(with_counts):
    mesh = plsc.VectorSubcoreMesh(core_axis_name="c", subcore_axis_name="s",
                                  num_cores=NC, num_subcores=NS)

    out_type = [
        jax.ShapeDtypeStruct((NC, N_PAD, H), jnp.float32),
        jax.ShapeDtypeStruct((NC, N_PAD, H), jnp.float32),
    ]
    scratch = [
        pltpu.VMEM_SHARED((N_PAD, H), jnp.float32),
        pltpu.VMEM_SHARED((N_PAD, H), jnp.float32),
        pltpu.VMEM_SHARED((N_PAD, H), jnp.float32),
        pltpu.VMEM((ROWS, CHUNK), jnp.int32),
        pltpu.VMEM((ROWS, CHUNK), jnp.int32),
        pltpu.VMEM((DEPTH, CHUNK, H), jnp.float32),
        [pltpu.SemaphoreType.DMA] * DEPTH,
        [pltpu.SemaphoreType.DMA] * DEPTH,
    ]
    if with_counts:
        out_type += [
            jax.ShapeDtypeStruct((NC, N_PAD, 16), jnp.float32),
            jax.ShapeDtypeStruct((NC, N_PAD, 16), jnp.float32),
        ]
        scratch += [
            pltpu.VMEM_SHARED((N_PAD, 16), jnp.float32),
            pltpu.VMEM_SHARED((N_PAD, 16), jnp.float32),
            pltpu.VMEM((CHUNK, 16), jnp.float32),
            [pltpu.SemaphoreType.DMA] * DEPTH,
        ]

    @functools.partial(
        pl.kernel, out_type=out_type, mesh=mesh, scratch_types=scratch,
        compiler_params=pltpu.CompilerParams(use_tc_tiling_on_sc=False))
    def k(*refs):
        if with_counts:
            (p_u2i, p_i2u, si_u2i, di_u2i, si_i2u, di_i2u, zeros_h, zeros_s,
             ones_h, agg_i_out, agg_u_out, cnt_i_out, cnt_u_out,
             acc_a, acc_b, tab, sidx, didx, rows, gsems, ssems,
             cnt_a, cnt_b, ones_v, osems) = refs
        else:
            (p_u2i, p_i2u, si_u2i, di_u2i, si_i2u, di_i2u, zeros_h,
             agg_i_out, agg_u_out,
             acc_a, acc_b, tab, sidx, didx, rows, gsems, ssems) = refs

        cid = lax.axis_index("c")
        sid = lax.axis_index("s")
        wid = sid * NC + cid

        gsem_a, gsem_b = gsems[0], gsems[1]
        ssem_a, ssem_b = ssems[0], ssems[1]
        if with_counts:
            osem_a, osem_b = osems[0], osems[1]

        zsl = pl.ds(sid * ZROWS, ZROWS)
        pltpu.sync_copy(zeros_h, acc_a.at[zsl])
        pltpu.sync_copy(zeros_h, acc_b.at[zsl])
        pltpu.sync_copy(p_u2i.at[zsl], tab.at[zsl])
        if with_counts:
            pltpu.sync_copy(zeros_s, cnt_a.at[zsl])
            pltpu.sync_copy(zeros_s, cnt_b.at[zsl])
            pltpu.sync_copy(ones_h, ones_v)
        plsc.subcore_barrier()

        def do_rel(p_tab, si_hbm, di_hbm, acc_sh, cnt_sh):
            isl = pl.ds(wid * ROWS, ROWS)
            pltpu.sync_copy(si_hbm.at[isl], sidx)
            pltpu.sync_copy(di_hbm.at[isl], didx)

            def g_issue(j, b, sem):
                pltpu.async_copy(p_tab.at[sidx.at[j]], rows.at[b], sem)

            def g_wait(b, sem):
                pltpu.make_async_copy(p_tab.at[sidx.at[0]], rows.at[b],
                                      sem).wait()

            def s_issue(j, b, sem):
                pltpu.async_copy(rows.at[b], acc_sh.at[didx.at[j]], sem,
                                 add=True)

            def s_wait(b, sem):
                pltpu.make_async_copy(rows.at[b], acc_sh.at[didx.at[0]],
                                      sem).wait()

            def o_issue(j, sem):
                pltpu.async_copy(ones_v, cnt_sh.at[didx.at[j]], sem, add=True)

            def o_wait(sem):
                pltpu.make_async_copy(ones_v, cnt_sh.at[didx.at[0]],
                                      sem).wait()

            half = ROWS // 2
            g_issue(0, 0, gsem_a)

            def pair(i, carry):
                j0 = 2 * i
                g_wait(0, gsem_a)

                @pl.when(i > 0)
                def _():
                    s_wait(1, ssem_b)
                    if cnt_sh is not None:
                        o_wait(osem_a)

                g_issue(j0 + 1, 1, gsem_b)
                s_issue(j0, 0, ssem_a)
                if cnt_sh is not None:
                    o_issue(j0, osem_a)

                g_wait(1, gsem_b)
                s_wait(0, ssem_a)

                if cnt_sh is not None:
                    @pl.when(i > 0)
                    def _():
                        o_wait(osem_b)

                @pl.when(i < half - 1)
                def _():
                    g_issue(j0 + 2, 0, gsem_a)

                s_issue(j0 + 1, 1, ssem_b)
                if cnt_sh is not None:
                    o_issue(j0 + 1, osem_b)
                return carry

            lax.fori_loop(0, half, pair, 0)
            s_wait(1, ssem_b)
            if cnt_sh is not None:
                o_wait(osem_a)
                o_wait(osem_b)

        do_rel(tab, si_u2i, di_u2i, acc_a, cnt_a if with_counts else None)
        plsc.subcore_barrier()
        pltpu.sync_copy(p_i2u.at[zsl], tab.at[zsl])
        plsc.subcore_barrier()
        do_rel(tab, si_i2u, di_i2u, acc_b, cnt_b if with_counts else None)
        plsc.subcore_barrier()

        pltpu.sync_copy(acc_a.at[zsl], agg_i_out.at[cid, zsl])
        pltpu.sync_copy(acc_b.at[zsl], agg_u_out.at[cid, zsl])
        if with_counts:
            pltpu.sync_copy(cnt_a.at[zsl], cnt_i_out.at[cid, zsl])
            pltpu.sync_copy(cnt_b.at[zsl], cnt_u_out.at[cid, zsl])

    return k


_edge_pass_l1 = _edge_pass(with_counts=True)
_edge_pass_l2 = _edge_pass(with_counts=False)


def _pad_rows(x):
    return jnp.pad(x, ((0, N_PAD - x.shape[0]), (0, 0)))


def _pad_idx(ix):
    pad = N_USER + jnp.arange(E_PAD - E, dtype=jnp.int32) % (N_PAD - N_USER)
    return jnp.concatenate([ix, pad]).reshape(NW * ROWS, CHUNK)


def kernel(x_user, x_item, edge_index_u2i, edge_index_i2u,
           W_l1_u2i, b_l1_u2i, W_r1_u2i,
           W_l1_i2u, b_l1_i2u, W_r1_i2u,
           W_l2_u2i, b_l2_u2i, W_r2_u2i,
           W_l2_i2u, b_l2_i2u, W_r2_i2u):
    si_u2i = _pad_idx(edge_index_u2i[0])
    di_u2i = _pad_idx(edge_index_u2i[1])
    si_i2u = _pad_idx(edge_index_i2u[0])
    di_i2u = _pad_idx(edge_index_i2u[1])
    zeros_h = jnp.zeros((ZROWS, H), jnp.float32)
    zeros_s = jnp.zeros((ZROWS, 16), jnp.float32)
    ones_h = jnp.ones((CHUNK, 16), jnp.float32)

    p1_u, s1_u, p1_i, s1_i = _proj4(
        _pad_rows(x_user), W_l1_u2i, W_r1_i2u,
        _pad_rows(x_item), W_l1_i2u, W_r1_u2i)

    agg1_i, agg1_u, cnt_i, cnt_u = _edge_pass_l1(
        p1_u, p1_i, si_u2i, di_u2i, si_i2u, di_i2u, zeros_h, zeros_s, ones_h)

    m2_i, s2_i = _combine_project(agg1_i, cnt_i, s1_i, b_l1_u2i,
                                  W_l2_i2u, W_r2_u2i)
    m2_u, s2_u = _combine_project(agg1_u, cnt_u, s1_u, b_l1_i2u,
                                  W_l2_u2i, W_r2_i2u)

    agg2_i, agg2_u = _edge_pass_l2(
        m2_u, m2_i, si_u2i, di_u2i, si_i2u, di_i2u, zeros_h)

    o_item = _combine_final(agg2_i, cnt_i, s2_i, b_l2_u2i, N_ITEM)
    o_user = _combine_final(agg2_u, cnt_u, s2_u, b_l2_i2u, N_USER)
    return (o_user, o_item)


# speedup vs baseline: 16.7391x; 1.2117x over previous
import functools

import jax
import jax.numpy as jnp
from jax import lax
from jax.experimental import pallas as pl
from jax.experimental.pallas import tpu as pltpu
from jax.experimental.pallas import tpu_sc as plsc

N_USER = 10000
N_ITEM = 10000
E = 320000
D_IN = 128
H = 32

NC = 2
NS = 16
NW = NC * NS
CHUNK = 128
ROWS = 80
DEPTH = 4
AHEAD = 2
E_PAD = NW * CHUNK * ROWS
N_PAD = 10240
ZROWS = N_PAD // NS


_VMEM100 = pltpu.CompilerParams(vmem_limit_bytes=100 * 1024 * 1024)


def _proj4_body(xu, xi, wa, wb, wc, wd, o1, o2, o3, o4):
    o1[...] = jnp.dot(xu[...], wa[...], preferred_element_type=jnp.float32)
    o2[...] = jnp.dot(xu[...], wb[...], preferred_element_type=jnp.float32)
    o3[...] = jnp.dot(xi[...], wc[...], preferred_element_type=jnp.float32)
    o4[...] = jnp.dot(xi[...], wd[...], preferred_element_type=jnp.float32)


def _proj4(xu, wa, wb, xi, wc, wd):
    return pl.pallas_call(
        _proj4_body,
        out_shape=[jax.ShapeDtypeStruct((N_PAD, H), jnp.float32)] * 4,
        compiler_params=_VMEM100,
    )(xu, xi, wa, wb, wc, wd)


def _combine(parts_ref, cnt_ref, r_ref, b_ref):
    agg = parts_ref[0] + parts_ref[1]
    cnt = cnt_ref[0, :, :1] + cnt_ref[1, :, :1]
    return jnp.maximum(agg / jnp.maximum(cnt, 1.0) + b_ref[...] + r_ref[...],
                       0.0)


def _mid_body(p, c, r, b, wl, wr, o1, o2):
    h = _combine(p, c, r, b)
    o1[...] = jnp.dot(h, wl[...], preferred_element_type=jnp.float32)
    o2[...] = jnp.dot(h, wr[...], preferred_element_type=jnp.float32)


def _fin_body(p, c, r, b, o_ref):
    o_ref[...] = _combine(p, c, r, b)[:o_ref.shape[0]]


def _combine_project(p, c, r, b, wl, wr):
    return pl.pallas_call(
        _mid_body,
        out_shape=[jax.ShapeDtypeStruct((N_PAD, H), jnp.float32)] * 2,
        compiler_params=_VMEM100,
    )(p, c, r, b.reshape(1, H), wl, wr)


def _combine_final(p, c, r, b, n):
    return pl.pallas_call(
        _fin_body,
        out_shape=jax.ShapeDtypeStruct((n, H), jnp.float32),
        compiler_params=_VMEM100,
    )(p, c, r, b.reshape(1, H))


def _edge_rel(with_counts):
    mesh = plsc.VectorSubcoreMesh(core_axis_name="c", subcore_axis_name="s",
                                  num_cores=NC, num_subcores=NS)

    out_type = [jax.ShapeDtypeStruct((NC, N_PAD, H), jnp.float32)]
    scratch = [
        pltpu.VMEM_SHARED((N_PAD, H), jnp.float32),
        pltpu.VMEM_SHARED((N_PAD, H), jnp.float32),
        pltpu.VMEM((ROWS, CHUNK), jnp.int32),
        pltpu.VMEM((ROWS, CHUNK), jnp.int32),
        pltpu.VMEM((DEPTH, CHUNK, H), jnp.float32),
        [pltpu.SemaphoreType.DMA] * DEPTH,
        [pltpu.SemaphoreType.DMA] * DEPTH,
    ]
    if with_counts:
        out_type += [jax.ShapeDtypeStruct((NC, N_PAD, 16), jnp.float32)]
        scratch += [
            pltpu.VMEM_SHARED((N_PAD, 16), jnp.float32),
            pltpu.VMEM((CHUNK, 16), jnp.float32),
            [pltpu.SemaphoreType.DMA] * DEPTH,
        ]

    @functools.partial(
        pl.kernel, out_type=out_type, mesh=mesh, scratch_types=scratch,
        compiler_params=pltpu.CompilerParams(use_tc_tiling_on_sc=False))
    def k(*refs):
        if with_counts:
            (p_hbm, si_hbm, di_hbm, zeros_h, zeros_s, ones_h,
             agg_out, cnt_out,
             acc, tab, sidx, didx, rows, gsems, ssems,
             cnt_sh, ones_v, osems) = refs
        else:
            (p_hbm, si_hbm, di_hbm, zeros_h,
             agg_out,
             acc, tab, sidx, didx, rows, gsems, ssems) = refs
            cnt_sh = None

        cid = lax.axis_index("c")
        sid = lax.axis_index("s")
        wid = sid * NC + cid

        gsem_a, gsem_b = gsems[0], gsems[1]
        ssem_a, ssem_b = ssems[0], ssems[1]
        if with_counts:
            osem_a, osem_b = osems[0], osems[1]

        zsl = pl.ds(sid * ZROWS, ZROWS)
        pltpu.sync_copy(zeros_h, acc.at[zsl])
        pltpu.sync_copy(p_hbm.at[zsl], tab.at[zsl])
        if with_counts:
            pltpu.sync_copy(zeros_s, cnt_sh.at[zsl])
            pltpu.sync_copy(ones_h, ones_v)

        isl = pl.ds(wid * ROWS, ROWS)
        pltpu.sync_copy(si_hbm.at[isl], sidx)
        pltpu.sync_copy(di_hbm.at[isl], didx)
        plsc.subcore_barrier()

        def g_issue(j, b, sem):
            pltpu.async_copy(tab.at[sidx.at[j]], rows.at[b], sem)

        def g_wait(b, sem):
            pltpu.make_async_copy(tab.at[sidx.at[0]], rows.at[b], sem).wait()

        def s_issue(j, b, sem):
            pltpu.async_copy(rows.at[b], acc.at[didx.at[j]], sem, add=True)

        def s_wait(b, sem):
            pltpu.make_async_copy(rows.at[b], acc.at[didx.at[0]], sem).wait()

        def o_issue(j, sem):
            pltpu.async_copy(ones_v, cnt_sh.at[didx.at[j]], sem, add=True)

        def o_wait(sem):
            pltpu.make_async_copy(ones_v, cnt_sh.at[didx.at[0]], sem).wait()

        half = ROWS // 2
        g_issue(0, 0, gsem_a)

        def pair(i, carry):
            j0 = 2 * i
            g_wait(0, gsem_a)

            @pl.when(i > 0)
            def _():
                s_wait(1, ssem_b)
                if cnt_sh is not None:
                    o_wait(osem_a)

            g_issue(j0 + 1, 1, gsem_b)
            s_issue(j0, 0, ssem_a)
            if cnt_sh is not None:
                o_issue(j0, osem_a)

            g_wait(1, gsem_b)
            s_wait(0, ssem_a)

            if cnt_sh is not None:
                @pl.when(i > 0)
                def _():
                    o_wait(osem_b)

            @pl.when(i < half - 1)
            def _():
                g_issue(j0 + 2, 0, gsem_a)

            s_issue(j0 + 1, 1, ssem_b)
            if cnt_sh is not None:
                o_issue(j0 + 1, osem_b)
            return carry

        lax.fori_loop(0, half, pair, 0)
        s_wait(1, ssem_b)
        if cnt_sh is not None:
            o_wait(osem_a)
            o_wait(osem_b)
        plsc.subcore_barrier()

        pltpu.sync_copy(acc.at[zsl], agg_out.at[cid, zsl])
        if with_counts:
            pltpu.sync_copy(cnt_sh.at[zsl], cnt_out.at[cid, zsl])

    return k


_edge_rel_cnt = _edge_rel(with_counts=True)
_edge_rel_nc = _edge_rel(with_counts=False)


def _pad_rows(x):
    return jnp.pad(x, ((0, N_PAD - x.shape[0]), (0, 0)))


def _pad_idx(ix):
    pad = N_USER + jnp.arange(E_PAD - E, dtype=jnp.int32) % (N_PAD - N_USER)
    return jnp.concatenate([ix, pad]).reshape(NW * ROWS, CHUNK)


def kernel(x_user, x_item, edge_index_u2i, edge_index_i2u,
           W_l1_u2i, b_l1_u2i, W_r1_u2i,
           W_l1_i2u, b_l1_i2u, W_r1_i2u,
           W_l2_u2i, b_l2_u2i, W_r2_u2i,
           W_l2_i2u, b_l2_i2u, W_r2_i2u):
    si_u2i = _pad_idx(edge_index_u2i[0])
    di_u2i = _pad_idx(edge_index_u2i[1])
    si_i2u = _pad_idx(edge_index_i2u[0])
    di_i2u = _pad_idx(edge_index_i2u[1])
    zeros_h = jnp.zeros((ZROWS, H), jnp.float32)
    zeros_s = jnp.zeros((ZROWS, 16), jnp.float32)
    ones_h = jnp.ones((CHUNK, 16), jnp.float32)

    p1_u, s1_u, p1_i, s1_i = _proj4(
        _pad_rows(x_user), W_l1_u2i, W_r1_i2u,
        _pad_rows(x_item), W_l1_i2u, W_r1_u2i)

    agg1_i, cnt_i = _edge_rel_cnt(p1_u, si_u2i, di_u2i,
                                  zeros_h, zeros_s, ones_h)
    agg1_u, cnt_u = _edge_rel_cnt(p1_i, si_i2u, di_i2u,
                                  zeros_h, zeros_s, ones_h)
    m2_i, s2_i = _combine_project(agg1_i, cnt_i, s1_i, b_l1_u2i,
                                  W_l2_i2u, W_r2_u2i)

    agg2_u = _edge_rel_nc(m2_i, si_i2u, di_i2u, zeros_h)[0]
    m2_u, s2_u = _combine_project(agg1_u, cnt_u, s1_u, b_l1_i2u,
                                  W_l2_u2i, W_r2_i2u)

    agg2_i = _edge_rel_nc(m2_u, si_u2i, di_u2i, zeros_h)[0]
    o_user = _combine_final(agg2_u, cnt_u, s2_u, b_l2_i2u, N_USER)
    o_item = _combine_final(agg2_i, cnt_i, s2_i, b_l2_u2i, N_ITEM)
    return (o_user, o_item)
